# baseline (speedup 1.0000x reference)
import jax
import jax.numpy as jnp
from jax.experimental import pallas as pl

def _dot(a, b, precision=None):
    return jax.lax.dot_general(
        a.astype(jnp.bfloat16), b.astype(jnp.bfloat16),
        (((a.ndim - 1,), (0,)), ((), ())),
        preferred_element_type=jnp.float32)

N = 8192
B = 8
K = 16


def _dgcnn_layer(x, batch, W1, b1, W2, b2, k=K):
    sq = jnp.sum(x * x, axis=-1)
    d2 = sq[:, None] + sq[None, :] - 2.0 * _dot(x, x.T)
    mask = batch[:, None] != batch[None, :]
    d2 = jnp.where(mask, jnp.inf, d2)
    _, idx = jax.lax.top_k(-d2, k)
    xj = x[idx]
    xi = jnp.broadcast_to(x[:, None, :], xj.shape)
    e = jnp.concatenate([xi, xj - xi], axis=-1)
    h = jax.nn.relu(_dot(e, W1) + b1)
    h = jax.nn.relu(_dot(h, W2) + b2)
    out = jnp.max(h, axis=1)
    return jax.nn.relu(out)


def _noop_kernel(x_ref, o_ref):
    o_ref[...] = x_ref[...]


def kernel(x, pos, batch, W1a, b1a, W2a, b2a, W1b, b1b, W2b, b2b, Wh1, bh1, Wh2, bh2, Wh3, bh3):
    x1 = _dgcnn_layer(x, batch, W1a, b1a, W2a, b2a)
    x2 = _dgcnn_layer(x1, batch, W1b, b1b, W2b, b2b)
    global_fea = jax.ops.segment_max(x2, batch, num_segments=B)
    global_fea = jnp.repeat(global_fea, 1024, axis=0)
    x4 = jnp.concatenate([x1, x2, global_fea], axis=-1)
    h = jax.nn.relu(_dot(x4, Wh1) + bh1)
    h = _dot(h, Wh2) + bh2
    fea = _dot(h, Wh3) + bh3
    score = jax.nn.sigmoid(fea)
    x4 = pl.pallas_call(
        _noop_kernel,
        out_shape=jax.ShapeDtypeStruct(x4.shape, x4.dtype),
    )(x4)
    return (x4, score)


# speedup vs baseline: 4.5201x; 4.5201x over previous
import functools

import jax
import jax.numpy as jnp
from jax.experimental import pallas as pl
from jax.experimental.pallas import tpu as pltpu
from jax.experimental.pallas import tpu_sc as plsc

N = 8192
B = 8
K = 16
NEG_INF = float("-inf")


def _mdot(a, b):
    return jax.lax.dot_general(
        a.astype(jnp.bfloat16), b.astype(jnp.bfloat16),
        (((a.ndim - 1,), (0,)), ((), ())),
        preferred_element_type=jnp.float32)


def _topk_body(x_ref, xt_ref, bat_ref, batt_ref, idx_ref, d2_ref):
    r = x_ref.shape[0]
    n = xt_ref.shape[1]
    xb = x_ref[...]
    xt = xt_ref[...]
    sqr = jnp.sum(xb * xb, axis=1, keepdims=True)
    sqc = jnp.sum(xt * xt, axis=0, keepdims=True)
    dot = _mdot(xb, xt)
    d2 = sqr + sqc - 2.0 * dot
    mask = bat_ref[...] != batt_ref[...]
    d2_ref[...] = jnp.where(mask, jnp.inf, d2)

    iota = jax.lax.broadcasted_iota(jnp.int32, (r, n), 1)
    iota_k = jax.lax.broadcasted_iota(jnp.int32, (r, K), 1)

    def body(t, idxs):
        d = d2_ref[...]
        m = jnp.min(d, axis=1, keepdims=True)
        cand = jnp.where(d <= m, iota, n)
        j = jnp.min(cand, axis=1, keepdims=True)
        d2_ref[...] = jnp.where(iota == j, jnp.inf, d)
        return jnp.where(iota_k == t, j, idxs)

    idx_ref[...] = jax.lax.fori_loop(
        0, K, body, jnp.zeros((r, K), jnp.int32))


def _topk(x, xt, bat, batt, block_rows, interpret=False):
    n, d = x.shape
    grid = n // block_rows
    return pl.pallas_call(
        _topk_body,
        grid=(grid,),
        in_specs=[
            pl.BlockSpec((block_rows, d), lambda b: (b, 0)),
            pl.BlockSpec((d, n), lambda b: (0, 0)),
            pl.BlockSpec((block_rows, 1), lambda b: (b, 0)),
            pl.BlockSpec((1, n), lambda b: (0, 0)),
        ],
        out_specs=pl.BlockSpec((block_rows, K), lambda b: (b, 0)),
        out_shape=jax.ShapeDtypeStruct((n, K), jnp.int32),
        scratch_shapes=[pltpu.VMEM((block_rows, n), jnp.float32)],
        interpret=interpret,
    )(x, xt, bat, batt)


def _sc_gather(src, idx_flat, interpret=False):
    return jnp.take(src, idx_flat[0], axis=0)


def _sc_gather_disabled(src, idx_flat, interpret=False):
    m = idx_flat.shape[1]
    dim = src.shape[1]
    window = 128
    mesh = plsc.VectorSubcoreMesh(
        core_axis_name="core", subcore_axis_name="subcore")

    @functools.partial(
        pl.kernel,
        out_type=jax.ShapeDtypeStruct((m, dim), src.dtype),
        mesh=mesh)
    def _k(x_hbm, i_hbm, o_hbm):
        def body(i_vmem, o_vmem):
            pltpu.sync_copy(x_hbm.at[i_vmem.at[0]], o_vmem)

        pltpu.emit_pipeline(
            body,
            grid=(m // window,),
            in_specs=[pl.BlockSpec((1, window), index_map=lambda i: (0, i))],
            out_specs=[pl.BlockSpec((window, dim), index_map=lambda i: (i, 0))],
            core_axis_name="subcore",
            dimension_semantics=(pltpu.PARALLEL,),
        )(i_hbm, o_hbm)

    return _k(src, idx_flat)


def _edge_body(x_ref, g_ref, w1_ref, b1_ref, w2_ref, b2_ref, out_ref, *, d):
    r = x_ref.shape[0]
    f2 = w2_ref.shape[1]
    xi = x_ref[...][:, :d]
    xj = g_ref[...].reshape(r, K, g_ref.shape[1])[:, :, :d]
    xi3 = jnp.broadcast_to(xi[:, None, :], (r, K, d))
    e = jnp.concatenate([xi3, xj - xi3], axis=-1)
    e = e.reshape(r * K, 2 * d)
    h = jnp.maximum(_mdot(e, w1_ref[...]) + b1_ref[...], 0.0)
    h = jnp.maximum(_mdot(h, w2_ref[...]) + b2_ref[...], 0.0)
    out = jnp.max(h.reshape(r, K, f2), axis=1)
    out_ref[...] = jnp.maximum(out, 0.0)


def _edge_seg_body(x_ref, g_ref, w1_ref, b1_ref, w2_ref, b2_ref, bat_ref,
                   out_ref, seg_ref, *, d):
    _edge_body(x_ref, g_ref, w1_ref, b1_ref, w2_ref, b2_ref, out_ref, d=d)

    @pl.when(pl.program_id(0) == 0)
    def _():
        seg_ref[...] = jnp.full_like(seg_ref, NEG_INF)

    x2 = out_ref[...]
    bat = bat_ref[...]
    for s in range(B):
        v = jnp.max(jnp.where(bat == s, x2, NEG_INF), axis=0,
                    keepdims=True)
        seg_ref[s:s + 1, :] = jnp.maximum(seg_ref[s:s + 1, :], v)


def _edge_mlp(x, g, w1, b1, w2, b2, bat, d, block_rows, interpret=False):
    n = x.shape[0]
    f1 = w1.shape[1]
    f2 = w2.shape[1]
    grid = n // block_rows
    b1r = b1.reshape(1, f1)
    b2r = b2.reshape(1, f2)
    return pl.pallas_call(
        functools.partial(_edge_seg_body, d=d),
        grid=(grid,),
        in_specs=[
            pl.BlockSpec((block_rows, x.shape[1]), lambda b: (b, 0)),
            pl.BlockSpec((block_rows * K, g.shape[1]), lambda b: (b, 0)),
            pl.BlockSpec(w1.shape, lambda b: (0, 0)),
            pl.BlockSpec(b1r.shape, lambda b: (0, 0)),
            pl.BlockSpec(w2.shape, lambda b: (0, 0)),
            pl.BlockSpec(b2r.shape, lambda b: (0, 0)),
            pl.BlockSpec((block_rows, 1), lambda b: (b, 0)),
        ],
        out_specs=[
            pl.BlockSpec((block_rows, f2), lambda b: (b, 0)),
            pl.BlockSpec((B, f2), lambda b: (0, 0)),
        ],
        out_shape=[
            jax.ShapeDtypeStruct((n, f2), jnp.float32),
            jax.ShapeDtypeStruct((B, f2), jnp.float32),
        ],
        interpret=interpret,
    )(x, g, w1, b1r, w2, b2r, bat)


def _head_body(x1_ref, x2_ref, g_ref, w1_ref, b1_ref, w2_ref, b2_ref,
               w3_ref, b3_ref, x4_ref, score_ref):
    r = x1_ref.shape[0]
    g = jnp.broadcast_to(g_ref[0], (r, g_ref.shape[2]))
    x4 = jnp.concatenate([x1_ref[...], x2_ref[...], g], axis=-1)
    x4_ref[...] = x4
    h = jnp.maximum(_mdot(x4, w1_ref[...]) + b1_ref[...], 0.0)
    h = _mdot(h, w2_ref[...]) + b2_ref[...]
    fea = _mdot(h, w3_ref[...]) + b3_ref[...]
    score_ref[...] = jax.nn.sigmoid(fea)


def _head(x1, x2, gseg3, wh1, bh1, wh2, bh2, wh3, bh3, block_rows,
          interpret=False):
    n = x1.shape[0]
    seg_rep = 1024
    b1r = bh1.reshape(1, -1)
    b2r = bh2.reshape(1, -1)
    b3r = bh3.reshape(1, -1)
    grid = n // block_rows
    return pl.pallas_call(
        _head_body,
        grid=(grid,),
        in_specs=[
            pl.BlockSpec((block_rows, x1.shape[1]), lambda b: (b, 0)),
            pl.BlockSpec((block_rows, x2.shape[1]), lambda b: (b, 0)),
            pl.BlockSpec((1, 1, gseg3.shape[2]),
                         lambda b: (b * block_rows // seg_rep, 0, 0)),
            pl.BlockSpec(wh1.shape, lambda b: (0, 0)),
            pl.BlockSpec(b1r.shape, lambda b: (0, 0)),
            pl.BlockSpec(wh2.shape, lambda b: (0, 0)),
            pl.BlockSpec(b2r.shape, lambda b: (0, 0)),
            pl.BlockSpec(wh3.shape, lambda b: (0, 0)),
            pl.BlockSpec(b3r.shape, lambda b: (0, 0)),
        ],
        out_specs=[
            pl.BlockSpec((block_rows, x1.shape[1] + x2.shape[1]
                          + gseg3.shape[2]), lambda b: (b, 0)),
            pl.BlockSpec((block_rows, 1), lambda b: (b, 0)),
        ],
        out_shape=[
            jax.ShapeDtypeStruct(
                (n, x1.shape[1] + x2.shape[1] + gseg3.shape[2]), jnp.float32),
            jax.ShapeDtypeStruct((n, 1), jnp.float32),
        ],
        interpret=interpret,
    )(x1, x2, gseg3, wh1, b1r, wh2, b2r, wh3, b3r)


def kernel(x, pos, batch, W1a, b1a, W2a, b2a, W1b, b1b, W2b, b2b,
           Wh1, bh1, Wh2, bh2, Wh3, bh3):
    del pos
    n = x.shape[0]
    bat = batch.astype(jnp.int32).reshape(n, 1)
    batt = batch.astype(jnp.int32).reshape(1, n)

    xpad = jnp.pad(x, ((0, 0), (0, 128 - x.shape[1])))
    idx_a = _topk(x, x.T, bat, batt, block_rows=256)
    ga = _sc_gather(xpad, idx_a.reshape(1, n * K))
    x1, _ = _edge_mlp(xpad, ga, W1a, b1a, W2a, b2a, bat, d=3,
                      block_rows=256)

    x1pad = jnp.pad(x1, ((0, 0), (0, 64)))
    idx_b = _topk(x1, x1.T, bat, batt, block_rows=256)
    gb = _sc_gather(x1pad, idx_b.reshape(1, n * K))
    x2, gseg = _edge_mlp(x1pad, gb, W1b, b1b, W2b, b2b, bat, d=64,
                         block_rows=256)

    x4, score = _head(x1, x2, gseg.reshape(B, 1, -1),
                      Wh1, bh1, Wh2, bh2, Wh3, bh3, block_rows=512)
    return (x4, score)


# speedup vs baseline: 8.4792x; 1.8759x over previous
import functools

import jax
import jax.numpy as jnp
from jax.experimental import pallas as pl
from jax.experimental.pallas import tpu as pltpu
from jax.experimental.pallas import tpu_sc as plsc

N = 8192
B = 8
K = 16
NEG_INF = float("-inf")


def _mdot(a, b):
    return jax.lax.dot_general(
        a.astype(jnp.bfloat16), b.astype(jnp.bfloat16),
        (((a.ndim - 1,), (0,)), ((), ())),
        preferred_element_type=jnp.float32)


W_WIN = 2560


def _extract_topk(d2_ref, idx_ref, r, width, off):
    iota = jax.lax.broadcasted_iota(jnp.int32, (r, width), 1)
    iota_k = jax.lax.broadcasted_iota(jnp.int32, (r, K), 1)

    def body(t, idxs):
        d = d2_ref[:, :width]
        m = jnp.min(d, axis=1, keepdims=True)
        cand = jnp.where(d <= m, iota, width)
        j = jnp.min(cand, axis=1, keepdims=True)
        d2_ref[:, :width] = jnp.where(iota == j, jnp.inf, d)
        return jnp.where(iota_k == t, j, idxs)

    idx_ref[...] = jax.lax.fori_loop(
        0, K, body, jnp.zeros((r, K), jnp.int32)) + off


def _topk_body(x_ref, xt_ref, bat_ref, batt_ref, idx_ref, d2_ref):
    r = x_ref.shape[0]
    n = xt_ref.shape[1]
    xb = x_ref[...]
    sqr = jnp.sum(xb * xb, axis=1, keepdims=True)
    bat = bat_ref[...]
    batt = batt_ref[...]
    iota_n = jax.lax.broadcasted_iota(jnp.int32, (1, n), 1)

    bmin = jnp.min(bat)
    bmax = jnp.max(bat)
    lo = jnp.min(jnp.where(batt == bmin, iota_n, n))
    hi = jnp.max(jnp.where(batt == bmax, iota_n, -1)) + 1
    lo_al = jnp.minimum((lo // 512) * 512, n - W_WIN)
    lo_al = pl.multiple_of(lo_al, 512)
    in_window = (hi - lo_al) <= W_WIN

    @pl.when(in_window)
    def _():
        xt = xt_ref[:, pl.ds(lo_al, W_WIN)]
        sqc = jnp.sum(xt * xt, axis=0, keepdims=True)
        dot = _mdot(xb, xt)
        d2 = sqr + sqc - 2.0 * dot
        mask = bat != batt_ref[:, pl.ds(lo_al, W_WIN)]
        d2_ref[:, :W_WIN] = jnp.where(mask, jnp.inf, d2)
        _extract_topk(d2_ref, idx_ref, r, W_WIN, lo_al)

    @pl.when(jnp.logical_not(in_window))
    def _():
        xt = xt_ref[...]
        sqc = jnp.sum(xt * xt, axis=0, keepdims=True)
        dot = _mdot(xb, xt)
        d2 = sqr + sqc - 2.0 * dot
        d2_ref[...] = jnp.where(bat != batt, jnp.inf, d2)
        _extract_topk(d2_ref, idx_ref, r, n, 0)


def _topk(x, xt, bat, batt, block_rows, interpret=False):
    n, d = x.shape
    grid = n // block_rows
    return pl.pallas_call(
        _topk_body,
        grid=(grid,),
        in_specs=[
            pl.BlockSpec((block_rows, d), lambda b: (b, 0)),
            pl.BlockSpec((d, n), lambda b: (0, 0)),
            pl.BlockSpec((block_rows, 1), lambda b: (b, 0)),
            pl.BlockSpec((1, n), lambda b: (0, 0)),
        ],
        out_specs=pl.BlockSpec((block_rows, K), lambda b: (b, 0)),
        out_shape=jax.ShapeDtypeStruct((n, K), jnp.int32),
        scratch_shapes=[pltpu.VMEM((block_rows, n), jnp.float32)],
        interpret=interpret,
    )(x, xt, bat, batt)


def _sc_gather(src, idx_flat, interpret=False):
    return jnp.take(src, idx_flat[0], axis=0)


def _sc_gather_disabled(src, idx_flat, interpret=False):
    m = idx_flat.shape[1]
    dim = src.shape[1]
    window = 128
    mesh = plsc.VectorSubcoreMesh(
        core_axis_name="core", subcore_axis_name="subcore")

    @functools.partial(
        pl.kernel,
        out_type=jax.ShapeDtypeStruct((m, dim), src.dtype),
        mesh=mesh)
    def _k(x_hbm, i_hbm, o_hbm):
        def body(i_vmem, o_vmem):
            pltpu.sync_copy(x_hbm.at[i_vmem.at[0]], o_vmem)

        pltpu.emit_pipeline(
            body,
            grid=(m // window,),
            in_specs=[pl.BlockSpec((1, window), index_map=lambda i: (0, i))],
            out_specs=[pl.BlockSpec((window, dim), index_map=lambda i: (i, 0))],
            core_axis_name="subcore",
            dimension_semantics=(pltpu.PARALLEL,),
        )(i_hbm, o_hbm)

    return _k(src, idx_flat)


def _edge_body(x_ref, g_ref, w1_ref, b1_ref, w2_ref, b2_ref, out_ref, *, d):
    r = x_ref.shape[0]
    f2 = w2_ref.shape[1]
    xi = x_ref[...][:, :d]
    xj = g_ref[...].reshape(r, K, g_ref.shape[1])[:, :, :d]
    xi3 = jnp.broadcast_to(xi[:, None, :], (r, K, d))
    e = jnp.concatenate([xi3, xj - xi3], axis=-1)
    e = e.reshape(r * K, 2 * d)
    h = jnp.maximum(_mdot(e, w1_ref[...]) + b1_ref[...], 0.0)
    h = jnp.maximum(_mdot(h, w2_ref[...]) + b2_ref[...], 0.0)
    out = jnp.max(h.reshape(r, K, f2), axis=1)
    out_ref[...] = jnp.maximum(out, 0.0)


def _edge_seg_body(x_ref, g_ref, w1_ref, b1_ref, w2_ref, b2_ref, bat_ref,
                   out_ref, seg_ref, *, d):
    _edge_body(x_ref, g_ref, w1_ref, b1_ref, w2_ref, b2_ref, out_ref, d=d)

    @pl.when(pl.program_id(0) == 0)
    def _():
        seg_ref[...] = jnp.full_like(seg_ref, NEG_INF)

    x2 = out_ref[...]
    bat = bat_ref[...]
    for s in range(B):
        v = jnp.max(jnp.where(bat == s, x2, NEG_INF), axis=0,
                    keepdims=True)
        seg_ref[s:s + 1, :] = jnp.maximum(seg_ref[s:s + 1, :], v)


def _edge_mlp(x, g, w1, b1, w2, b2, bat, d, block_rows, interpret=False):
    n = x.shape[0]
    f1 = w1.shape[1]
    f2 = w2.shape[1]
    grid = n // block_rows
    b1r = b1.reshape(1, f1)
    b2r = b2.reshape(1, f2)
    return pl.pallas_call(
        functools.partial(_edge_seg_body, d=d),
        grid=(grid,),
        in_specs=[
            pl.BlockSpec((block_rows, x.shape[1]), lambda b: (b, 0)),
            pl.BlockSpec((block_rows * K, g.shape[1]), lambda b: (b, 0)),
            pl.BlockSpec(w1.shape, lambda b: (0, 0)),
            pl.BlockSpec(b1r.shape, lambda b: (0, 0)),
            pl.BlockSpec(w2.shape, lambda b: (0, 0)),
            pl.BlockSpec(b2r.shape, lambda b: (0, 0)),
            pl.BlockSpec((block_rows, 1), lambda b: (b, 0)),
        ],
        out_specs=[
            pl.BlockSpec((block_rows, f2), lambda b: (b, 0)),
            pl.BlockSpec((B, f2), lambda b: (0, 0)),
        ],
        out_shape=[
            jax.ShapeDtypeStruct((n, f2), jnp.float32),
            jax.ShapeDtypeStruct((B, f2), jnp.float32),
        ],
        interpret=interpret,
    )(x, g, w1, b1r, w2, b2r, bat)


def _head_body(x1_ref, x2_ref, g_ref, w1_ref, b1_ref, w2_ref, b2_ref,
               w3_ref, b3_ref, x4_ref, score_ref):
    r = x1_ref.shape[0]
    g = jnp.broadcast_to(g_ref[0], (r, g_ref.shape[2]))
    x4 = jnp.concatenate([x1_ref[...], x2_ref[...], g], axis=-1)
    x4_ref[...] = x4
    h = jnp.maximum(_mdot(x4, w1_ref[...]) + b1_ref[...], 0.0)
    h = _mdot(h, w2_ref[...]) + b2_ref[...]
    fea = _mdot(h, w3_ref[...]) + b3_ref[...]
    score_ref[...] = jax.nn.sigmoid(fea)


def _head(x1, x2, gseg3, wh1, bh1, wh2, bh2, wh3, bh3, block_rows,
          interpret=False):
    n = x1.shape[0]
    seg_rep = 1024
    b1r = bh1.reshape(1, -1)
    b2r = bh2.reshape(1, -1)
    b3r = bh3.reshape(1, -1)
    grid = n // block_rows
    return pl.pallas_call(
        _head_body,
        grid=(grid,),
        in_specs=[
            pl.BlockSpec((block_rows, x1.shape[1]), lambda b: (b, 0)),
            pl.BlockSpec((block_rows, x2.shape[1]), lambda b: (b, 0)),
            pl.BlockSpec((1, 1, gseg3.shape[2]),
                         lambda b: (b * block_rows // seg_rep, 0, 0)),
            pl.BlockSpec(wh1.shape, lambda b: (0, 0)),
            pl.BlockSpec(b1r.shape, lambda b: (0, 0)),
            pl.BlockSpec(wh2.shape, lambda b: (0, 0)),
            pl.BlockSpec(b2r.shape, lambda b: (0, 0)),
            pl.BlockSpec(wh3.shape, lambda b: (0, 0)),
            pl.BlockSpec(b3r.shape, lambda b: (0, 0)),
        ],
        out_specs=[
            pl.BlockSpec((block_rows, x1.shape[1] + x2.shape[1]
                          + gseg3.shape[2]), lambda b: (b, 0)),
            pl.BlockSpec((block_rows, 1), lambda b: (b, 0)),
        ],
        out_shape=[
            jax.ShapeDtypeStruct(
                (n, x1.shape[1] + x2.shape[1] + gseg3.shape[2]), jnp.float32),
            jax.ShapeDtypeStruct((n, 1), jnp.float32),
        ],
        interpret=interpret,
    )(x1, x2, gseg3, wh1, b1r, wh2, b2r, wh3, b3r)


def kernel(x, pos, batch, W1a, b1a, W2a, b2a, W1b, b1b, W2b, b2b,
           Wh1, bh1, Wh2, bh2, Wh3, bh3):
    del pos
    n = x.shape[0]
    bat = batch.astype(jnp.int32).reshape(n, 1)
    batt = batch.astype(jnp.int32).reshape(1, n)

    xpad = jnp.pad(x, ((0, 0), (0, 128 - x.shape[1])))
    idx_a = _topk(x, x.T, bat, batt, block_rows=256)
    ga = _sc_gather(xpad, idx_a.reshape(1, n * K))
    x1, _ = _edge_mlp(xpad, ga, W1a, b1a, W2a, b2a, bat, d=3,
                      block_rows=256)

    x1pad = jnp.pad(x1, ((0, 0), (0, 64)))
    idx_b = _topk(x1, x1.T, bat, batt, block_rows=256)
    gb = _sc_gather(x1pad, idx_b.reshape(1, n * K))
    x2, gseg = _edge_mlp(x1pad, gb, W1b, b1b, W2b, b2b, bat, d=64,
                         block_rows=256)

    x4, score = _head(x1, x2, gseg.reshape(B, 1, -1),
                      Wh1, bh1, Wh2, bh2, Wh3, bh3, block_rows=512)
    return (x4, score)


# speedup vs baseline: 14.3632x; 1.6939x over previous
import functools

import jax
import jax.numpy as jnp
from jax.experimental import pallas as pl
from jax.experimental.pallas import tpu as pltpu
from jax.experimental.pallas import tpu_sc as plsc

N = 8192
B = 8
K = 16
NEG_INF = float("-inf")


def _mdot(a, b):
    return jax.lax.dot_general(
        a.astype(jnp.bfloat16), b.astype(jnp.bfloat16),
        (((a.ndim - 1,), (0,)), ((), ())),
        preferred_element_type=jnp.float32)


W_WIN = 2560


def _extract_topk(d2_ref, idx_ref, r, width, off):
    iota = jax.lax.broadcasted_iota(jnp.int32, (r, width), 1)
    iota_k = jax.lax.broadcasted_iota(jnp.int32, (r, K), 1)

    def body(t, idxs):
        d = d2_ref[:, :width]
        m = jnp.min(d, axis=1, keepdims=True)
        cand = jnp.where(d <= m, iota, width)
        j = jnp.min(cand, axis=1, keepdims=True)
        d2_ref[:, :width] = jnp.where(iota == j, jnp.inf, d)
        return jnp.where(iota_k == t, j, idxs)

    idx_ref[...] = jax.lax.fori_loop(
        0, K, body, jnp.zeros((r, K), jnp.int32)) + off


def _topk_body(x_ref, xt_ref, bat_ref, batt_ref, idx_ref, d2_ref):
    r = x_ref.shape[0]
    n = xt_ref.shape[1]
    xb = x_ref[...]
    sqr = jnp.sum(xb * xb, axis=1, keepdims=True)
    bat = bat_ref[...]
    batt = batt_ref[...]
    iota_n = jax.lax.broadcasted_iota(jnp.int32, (1, n), 1)

    bmin = jnp.min(bat)
    bmax = jnp.max(bat)
    lo = jnp.min(jnp.where(batt == bmin, iota_n, n))
    hi = jnp.max(jnp.where(batt == bmax, iota_n, -1)) + 1
    lo_al = jnp.minimum((lo // 512) * 512, n - W_WIN)
    lo_al = pl.multiple_of(lo_al, 512)
    in_window = (hi - lo_al) <= W_WIN

    @pl.when(in_window)
    def _():
        xt = xt_ref[:, pl.ds(lo_al, W_WIN)]
        sqc = jnp.sum(xt * xt, axis=0, keepdims=True)
        dot = _mdot(xb, xt)
        d2 = sqr + sqc - 2.0 * dot
        mask = bat != batt_ref[:, pl.ds(lo_al, W_WIN)]
        d2_ref[:, :W_WIN] = jnp.where(mask, jnp.inf, d2)
        _extract_topk(d2_ref, idx_ref, r, W_WIN, lo_al)

    @pl.when(jnp.logical_not(in_window))
    def _():
        xt = xt_ref[...]
        sqc = jnp.sum(xt * xt, axis=0, keepdims=True)
        dot = _mdot(xb, xt)
        d2 = sqr + sqc - 2.0 * dot
        d2_ref[...] = jnp.where(bat != batt, jnp.inf, d2)
        _extract_topk(d2_ref, idx_ref, r, n, 0)


def _topk(x, xt, bat, batt, block_rows, interpret=False):
    n, d = x.shape
    grid = n // block_rows
    return pl.pallas_call(
        _topk_body,
        grid=(grid,),
        in_specs=[
            pl.BlockSpec((block_rows, d), lambda b: (b, 0)),
            pl.BlockSpec((d, n), lambda b: (0, 0)),
            pl.BlockSpec((block_rows, 1), lambda b: (b, 0)),
            pl.BlockSpec((1, n), lambda b: (0, 0)),
        ],
        out_specs=pl.BlockSpec((block_rows, K), lambda b: (b, 0)),
        out_shape=jax.ShapeDtypeStruct((n, K), jnp.int32),
        scratch_shapes=[pltpu.VMEM((block_rows, n), jnp.float32)],
        interpret=interpret,
    )(x, xt, bat, batt)


def _tc_gather(src, idx_flat, interpret=False):
    return jnp.take(src, idx_flat[0], axis=0)


def _sc_gather(src, idx_flat, window=128, interpret=False):
    m = idx_flat.shape[1]
    dim = src.shape[1]
    mesh = plsc.VectorSubcoreMesh(
        core_axis_name="core", subcore_axis_name="subcore")

    @functools.partial(
        pl.kernel,
        out_type=jax.ShapeDtypeStruct((m, dim), src.dtype),
        mesh=mesh)
    def _k(x_hbm, i_hbm, o_hbm):
        def body(i_vmem, o_vmem):
            pltpu.sync_copy(x_hbm.at[i_vmem.at[0]], o_vmem)

        pltpu.emit_pipeline(
            body,
            grid=(m // window,),
            in_specs=[pl.BlockSpec((1, window), index_map=lambda i: (0, i))],
            out_specs=[pl.BlockSpec((window, dim), index_map=lambda i: (i, 0))],
            core_axis_name=("core", "subcore"),
            dimension_semantics=(pltpu.PARALLEL,),
        )(i_hbm, o_hbm)

    return _k(src, idx_flat)


def _edge_body(x_ref, g_ref, w1_ref, b1_ref, w2_ref, b2_ref, out_ref, *, d):
    r = x_ref.shape[0]
    f2 = w2_ref.shape[1]
    xi = x_ref[...][:, :d]
    xj = g_ref[...].reshape(r, K, g_ref.shape[1])[:, :, :d]
    xi3 = jnp.broadcast_to(xi[:, None, :], (r, K, d))
    e = jnp.concatenate([xi3, xj - xi3], axis=-1)
    e = e.reshape(r * K, 2 * d)
    h = jnp.maximum(_mdot(e, w1_ref[...]) + b1_ref[...], 0.0)
    h = jnp.maximum(_mdot(h, w2_ref[...]) + b2_ref[...], 0.0)
    out = jnp.max(h.reshape(r, K, f2), axis=1)
    out_ref[...] = jnp.maximum(out, 0.0)


def _edge_seg_body(x_ref, g_ref, w1_ref, b1_ref, w2_ref, b2_ref, bat_ref,
                   out_ref, seg_ref, *, d):
    _edge_body(x_ref, g_ref, w1_ref, b1_ref, w2_ref, b2_ref, out_ref, d=d)

    @pl.when(pl.program_id(0) == 0)
    def _():
        seg_ref[...] = jnp.full_like(seg_ref, NEG_INF)

    x2 = out_ref[...]
    bat = bat_ref[...]
    for s in range(B):
        v = jnp.max(jnp.where(bat == s, x2, NEG_INF), axis=0,
                    keepdims=True)
        seg_ref[s:s + 1, :] = jnp.maximum(seg_ref[s:s + 1, :], v)


def _edge_mlp(x, g, w1, b1, w2, b2, bat, d, block_rows, interpret=False):
    n = x.shape[0]
    f1 = w1.shape[1]
    f2 = w2.shape[1]
    grid = n // block_rows
    b1r = b1.reshape(1, f1)
    b2r = b2.reshape(1, f2)
    return pl.pallas_call(
        functools.partial(_edge_seg_body, d=d),
        grid=(grid,),
        in_specs=[
            pl.BlockSpec((block_rows, x.shape[1]), lambda b: (b, 0)),
            pl.BlockSpec((block_rows * K, g.shape[1]), lambda b: (b, 0)),
            pl.BlockSpec(w1.shape, lambda b: (0, 0)),
            pl.BlockSpec(b1r.shape, lambda b: (0, 0)),
            pl.BlockSpec(w2.shape, lambda b: (0, 0)),
            pl.BlockSpec(b2r.shape, lambda b: (0, 0)),
            pl.BlockSpec((block_rows, 1), lambda b: (b, 0)),
        ],
        out_specs=[
            pl.BlockSpec((block_rows, f2), lambda b: (b, 0)),
            pl.BlockSpec((B, f2), lambda b: (0, 0)),
        ],
        out_shape=[
            jax.ShapeDtypeStruct((n, f2), jnp.float32),
            jax.ShapeDtypeStruct((B, f2), jnp.float32),
        ],
        interpret=interpret,
    )(x, g, w1, b1r, w2, b2r, bat)


def _head_body(x1_ref, x2_ref, g_ref, w1_ref, b1_ref, w2_ref, b2_ref,
               w3_ref, b3_ref, x4_ref, score_ref):
    r = x1_ref.shape[0]
    g = jnp.broadcast_to(g_ref[0], (r, g_ref.shape[2]))
    x4 = jnp.concatenate([x1_ref[...], x2_ref[...], g], axis=-1)
    x4_ref[...] = x4
    h = jnp.maximum(_mdot(x4, w1_ref[...]) + b1_ref[...], 0.0)
    h = _mdot(h, w2_ref[...]) + b2_ref[...]
    fea = _mdot(h, w3_ref[...]) + b3_ref[...]
    score_ref[...] = jax.nn.sigmoid(fea)


def _head(x1, x2, gseg3, wh1, bh1, wh2, bh2, wh3, bh3, block_rows,
          interpret=False):
    n = x1.shape[0]
    seg_rep = 1024
    b1r = bh1.reshape(1, -1)
    b2r = bh2.reshape(1, -1)
    b3r = bh3.reshape(1, -1)
    grid = n // block_rows
    return pl.pallas_call(
        _head_body,
        grid=(grid,),
        in_specs=[
            pl.BlockSpec((block_rows, x1.shape[1]), lambda b: (b, 0)),
            pl.BlockSpec((block_rows, x2.shape[1]), lambda b: (b, 0)),
            pl.BlockSpec((1, 1, gseg3.shape[2]),
                         lambda b: (b * block_rows // seg_rep, 0, 0)),
            pl.BlockSpec(wh1.shape, lambda b: (0, 0)),
            pl.BlockSpec(b1r.shape, lambda b: (0, 0)),
            pl.BlockSpec(wh2.shape, lambda b: (0, 0)),
            pl.BlockSpec(b2r.shape, lambda b: (0, 0)),
            pl.BlockSpec(wh3.shape, lambda b: (0, 0)),
            pl.BlockSpec(b3r.shape, lambda b: (0, 0)),
        ],
        out_specs=[
            pl.BlockSpec((block_rows, x1.shape[1] + x2.shape[1]
                          + gseg3.shape[2]), lambda b: (b, 0)),
            pl.BlockSpec((block_rows, 1), lambda b: (b, 0)),
        ],
        out_shape=[
            jax.ShapeDtypeStruct(
                (n, x1.shape[1] + x2.shape[1] + gseg3.shape[2]), jnp.float32),
            jax.ShapeDtypeStruct((n, 1), jnp.float32),
        ],
        interpret=interpret,
    )(x1, x2, gseg3, wh1, b1r, wh2, b2r, wh3, b3r)


def kernel(x, pos, batch, W1a, b1a, W2a, b2a, W1b, b1b, W2b, b2b,
           Wh1, bh1, Wh2, bh2, Wh3, bh3):
    del pos
    n = x.shape[0]
    bat = batch.astype(jnp.int32).reshape(n, 1)
    batt = batch.astype(jnp.int32).reshape(1, n)

    xpad = jnp.pad(x, ((0, 0), (0, 128 - x.shape[1])))
    idx_a = _topk(x, x.T, bat, batt, block_rows=256)
    ga = _sc_gather(xpad, idx_a.reshape(1, n * K))
    x1, _ = _edge_mlp(xpad, ga, W1a, b1a, W2a, b2a, bat, d=3,
                      block_rows=256)

    x1pad = jnp.pad(x1, ((0, 0), (0, 64)))
    idx_b = _topk(x1, x1.T, bat, batt, block_rows=256)
    gb = _sc_gather(x1pad, idx_b.reshape(1, n * K), window=256)
    x2, gseg = _edge_mlp(x1pad, gb, W1b, b1b, W2b, b2b, bat, d=64,
                         block_rows=256)

    x4, score = _head(x1, x2, gseg.reshape(B, 1, -1),
                      Wh1, bh1, Wh2, bh2, Wh3, bh3, block_rows=512)
    return (x4, score)


# speedup vs baseline: 17.4827x; 1.2172x over previous
import functools

import jax
import jax.numpy as jnp
from jax.experimental import pallas as pl
from jax.experimental.pallas import tpu as pltpu
from jax.experimental.pallas import tpu_sc as plsc

N = 8192
B = 8
K = 16
NEG_INF = float("-inf")


def _mdot(a, b):
    return jax.lax.dot_general(
        a.astype(jnp.bfloat16), b.astype(jnp.bfloat16),
        (((a.ndim - 1,), (0,)), ((), ())),
        preferred_element_type=jnp.float32)


W_WIN1 = 1536
W_WIN2 = 2560


def _extract_topk(d2_ref, idx_ref, r, width, off):
    iota = jax.lax.broadcasted_iota(jnp.int32, (r, width), 1)
    iota_k = jax.lax.broadcasted_iota(jnp.int32, (r, K), 1)

    def body(t, idxs):
        d = d2_ref[:, :width]
        m = jnp.min(d, axis=1, keepdims=True)
        cand = jnp.where(d <= m, iota, width)
        j = jnp.min(cand, axis=1, keepdims=True)
        d2_ref[:, :width] = jnp.where(iota == j, jnp.inf, d)
        return jnp.where(iota_k == t, j, idxs)

    idx_ref[...] = jax.lax.fori_loop(
        0, K, body, jnp.zeros((r, K), jnp.int32)) + off


def _topk_body(x_ref, xt_ref, bat_ref, batt_ref, idx_ref, d2_ref):
    r = x_ref.shape[0]
    n = xt_ref.shape[1]
    xb = x_ref[...]
    sqr = jnp.sum(xb * xb, axis=1, keepdims=True)
    bat = bat_ref[...]
    batt = batt_ref[...]
    iota_n = jax.lax.broadcasted_iota(jnp.int32, (1, n), 1)

    bmin = jnp.min(bat)
    bmax = jnp.max(bat)
    lo = jnp.min(jnp.where(batt == bmin, iota_n, n))
    hi = jnp.max(jnp.where(batt == bmax, iota_n, -1)) + 1

    def _windowed(width):
        lo_al = jnp.minimum((lo // 256) * 256, n - width)
        lo_al = pl.multiple_of(lo_al, 256)
        return lo_al, (hi - lo_al) <= width

    lo1, fits1 = _windowed(W_WIN1)
    lo2, fits2 = _windowed(W_WIN2)

    def _run(width, lo_al):
        xt = xt_ref[:, pl.ds(lo_al, width)]
        sqc = jnp.sum(xt * xt, axis=0, keepdims=True)
        dot = _mdot(xb, xt)
        d2 = sqr + sqc - 2.0 * dot
        mask = bat != batt_ref[:, pl.ds(lo_al, width)]
        d2_ref[:, :width] = jnp.where(mask, jnp.inf, d2)
        _extract_topk(d2_ref, idx_ref, r, width, lo_al)

    @pl.when(fits1)
    def _():
        _run(W_WIN1, lo1)

    @pl.when(jnp.logical_not(fits1) & fits2)
    def _():
        _run(W_WIN2, lo2)

    @pl.when(jnp.logical_not(fits2))
    def _():
        xt = xt_ref[...]
        sqc = jnp.sum(xt * xt, axis=0, keepdims=True)
        dot = _mdot(xb, xt)
        d2 = sqr + sqc - 2.0 * dot
        d2_ref[...] = jnp.where(bat != batt, jnp.inf, d2)
        _extract_topk(d2_ref, idx_ref, r, n, 0)


def _topk(x, xt, bat, batt, block_rows, interpret=False):
    n, d = x.shape
    grid = n // block_rows
    return pl.pallas_call(
        _topk_body,
        grid=(grid,),
        in_specs=[
            pl.BlockSpec((block_rows, d), lambda b: (b, 0)),
            pl.BlockSpec((d, n), lambda b: (0, 0)),
            pl.BlockSpec((block_rows, 1), lambda b: (b, 0)),
            pl.BlockSpec((1, n), lambda b: (0, 0)),
        ],
        out_specs=pl.BlockSpec((block_rows, K), lambda b: (b, 0)),
        out_shape=jax.ShapeDtypeStruct((n, K), jnp.int32),
        scratch_shapes=[pltpu.VMEM((block_rows, n), jnp.float32)],
        interpret=interpret,
    )(x, xt, bat, batt)


def _tc_gather(src, idx_flat, interpret=False):
    return jnp.take(src, idx_flat[0], axis=0)


def _sc_gather(src, idx_flat, window=128, interpret=False):
    m = idx_flat.shape[1]
    dim = src.shape[1]
    mesh = plsc.VectorSubcoreMesh(
        core_axis_name="core", subcore_axis_name="subcore")

    @functools.partial(
        pl.kernel,
        out_type=jax.ShapeDtypeStruct((m, dim), src.dtype),
        mesh=mesh)
    def _k(x_hbm, i_hbm, o_hbm):
        def body(i_vmem, o_vmem):
            pltpu.sync_copy(x_hbm.at[i_vmem.at[0]], o_vmem)

        pltpu.emit_pipeline(
            body,
            grid=(m // window,),
            in_specs=[pl.BlockSpec((1, window), index_map=lambda i: (0, i))],
            out_specs=[pl.BlockSpec((window, dim), index_map=lambda i: (i, 0))],
            core_axis_name=("core", "subcore"),
            dimension_semantics=(pltpu.PARALLEL,),
        )(i_hbm, o_hbm)

    return _k(src, idx_flat)


def _edge_body(x_ref, g_ref, w1_ref, b1_ref, w2_ref, b2_ref, out_ref, *, d):
    r = x_ref.shape[0]
    f2 = w2_ref.shape[1]
    xi = x_ref[...][:, :d]
    xj = g_ref[...].reshape(r, K, g_ref.shape[1])[:, :, :d]
    xi3 = jnp.broadcast_to(xi[:, None, :], (r, K, d))
    e = jnp.concatenate([xi3, xj - xi3], axis=-1)
    e = e.reshape(r * K, 2 * d)
    h = jnp.maximum(_mdot(e, w1_ref[...]) + b1_ref[...], 0.0)
    h = jnp.maximum(_mdot(h, w2_ref[...]) + b2_ref[...], 0.0)
    out = jnp.max(h.reshape(r, K, f2), axis=1)
    out_ref[...] = jnp.maximum(out, 0.0)


def _edge_seg_body(x_ref, g_ref, w1_ref, b1_ref, w2_ref, b2_ref, bat_ref,
                   out_ref, seg_ref, *, d):
    _edge_body(x_ref, g_ref, w1_ref, b1_ref, w2_ref, b2_ref, out_ref, d=d)

    @pl.when(pl.program_id(0) == 0)
    def _():
        seg_ref[...] = jnp.full_like(seg_ref, NEG_INF)

    x2 = out_ref[...]
    bat = bat_ref[...]
    for s in range(B):
        v = jnp.max(jnp.where(bat == s, x2, NEG_INF), axis=0,
                    keepdims=True)
        seg_ref[s:s + 1, :] = jnp.maximum(seg_ref[s:s + 1, :], v)


def _edge_mlp(x, g, w1, b1, w2, b2, bat, d, block_rows, interpret=False):
    n = x.shape[0]
    f1 = w1.shape[1]
    f2 = w2.shape[1]
    grid = n // block_rows
    b1r = b1.reshape(1, f1)
    b2r = b2.reshape(1, f2)
    return pl.pallas_call(
        functools.partial(_edge_seg_body, d=d),
        grid=(grid,),
        in_specs=[
            pl.BlockSpec((block_rows, x.shape[1]), lambda b: (b, 0)),
            pl.BlockSpec((block_rows * K, g.shape[1]), lambda b: (b, 0)),
            pl.BlockSpec(w1.shape, lambda b: (0, 0)),
            pl.BlockSpec(b1r.shape, lambda b: (0, 0)),
            pl.BlockSpec(w2.shape, lambda b: (0, 0)),
            pl.BlockSpec(b2r.shape, lambda b: (0, 0)),
            pl.BlockSpec((block_rows, 1), lambda b: (b, 0)),
        ],
        out_specs=[
            pl.BlockSpec((block_rows, f2), lambda b: (b, 0)),
            pl.BlockSpec((B, f2), lambda b: (0, 0)),
        ],
        out_shape=[
            jax.ShapeDtypeStruct((n, f2), jnp.float32),
            jax.ShapeDtypeStruct((B, f2), jnp.float32),
        ],
        interpret=interpret,
    )(x, g, w1, b1r, w2, b2r, bat)


def _head_body(x1_ref, x2_ref, g_ref, w1_ref, b1_ref, w2_ref, b2_ref,
               w3_ref, b3_ref, x4_ref, score_ref):
    r = x1_ref.shape[0]
    g = jnp.broadcast_to(g_ref[0], (r, g_ref.shape[2]))
    x4 = jnp.concatenate([x1_ref[...], x2_ref[...], g], axis=-1)
    x4_ref[...] = x4
    h = jnp.maximum(_mdot(x4, w1_ref[...]) + b1_ref[...], 0.0)
    h = _mdot(h, w2_ref[...]) + b2_ref[...]
    fea = _mdot(h, w3_ref[...]) + b3_ref[...]
    score_ref[...] = jax.nn.sigmoid(fea)


def _head(x1, x2, gseg3, wh1, bh1, wh2, bh2, wh3, bh3, block_rows,
          interpret=False):
    n = x1.shape[0]
    seg_rep = 1024
    b1r = bh1.reshape(1, -1)
    b2r = bh2.reshape(1, -1)
    b3r = bh3.reshape(1, -1)
    grid = n // block_rows
    return pl.pallas_call(
        _head_body,
        grid=(grid,),
        in_specs=[
            pl.BlockSpec((block_rows, x1.shape[1]), lambda b: (b, 0)),
            pl.BlockSpec((block_rows, x2.shape[1]), lambda b: (b, 0)),
            pl.BlockSpec((1, 1, gseg3.shape[2]),
                         lambda b: (b * block_rows // seg_rep, 0, 0)),
            pl.BlockSpec(wh1.shape, lambda b: (0, 0)),
            pl.BlockSpec(b1r.shape, lambda b: (0, 0)),
            pl.BlockSpec(wh2.shape, lambda b: (0, 0)),
            pl.BlockSpec(b2r.shape, lambda b: (0, 0)),
            pl.BlockSpec(wh3.shape, lambda b: (0, 0)),
            pl.BlockSpec(b3r.shape, lambda b: (0, 0)),
        ],
        out_specs=[
            pl.BlockSpec((block_rows, x1.shape[1] + x2.shape[1]
                          + gseg3.shape[2]), lambda b: (b, 0)),
            pl.BlockSpec((block_rows, 1), lambda b: (b, 0)),
        ],
        out_shape=[
            jax.ShapeDtypeStruct(
                (n, x1.shape[1] + x2.shape[1] + gseg3.shape[2]), jnp.float32),
            jax.ShapeDtypeStruct((n, 1), jnp.float32),
        ],
        interpret=interpret,
    )(x1, x2, gseg3, wh1, b1r, wh2, b2r, wh3, b3r)


def kernel(x, pos, batch, W1a, b1a, W2a, b2a, W1b, b1b, W2b, b2b,
           Wh1, bh1, Wh2, bh2, Wh3, bh3):
    del pos
    n = x.shape[0]
    bat = batch.astype(jnp.int32).reshape(n, 1)
    batt = batch.astype(jnp.int32).reshape(1, n)

    xpad = jnp.pad(x, ((0, 0), (0, 128 - x.shape[1])))
    idx_a = _topk(x, x.T, bat, batt, block_rows=256)
    ga = _sc_gather(xpad, idx_a.reshape(1, n * K))
    x1, _ = _edge_mlp(xpad, ga, W1a, b1a, W2a, b2a, bat, d=3,
                      block_rows=256)

    x1pad = jnp.pad(x1, ((0, 0), (0, 64)))
    idx_b = _topk(x1, x1.T, bat, batt, block_rows=256)
    gb = _sc_gather(x1pad, idx_b.reshape(1, n * K), window=256)
    x2, gseg = _edge_mlp(x1pad, gb, W1b, b1b, W2b, b2b, bat, d=64,
                         block_rows=256)

    x4, score = _head(x1, x2, gseg.reshape(B, 1, -1),
                      Wh1, bh1, Wh2, bh2, Wh3, bh3, block_rows=512)
    return (x4, score)


# speedup vs baseline: 17.6540x; 1.0098x over previous
import functools

import jax
import jax.numpy as jnp
from jax.experimental import pallas as pl
from jax.experimental.pallas import tpu as pltpu
from jax.experimental.pallas import tpu_sc as plsc

N = 8192
B = 8
K = 16
NEG_INF = float("-inf")


def _mdot(a, b):
    return jax.lax.dot_general(
        a.astype(jnp.bfloat16), b.astype(jnp.bfloat16),
        (((a.ndim - 1,), (0,)), ((), ())),
        preferred_element_type=jnp.float32)


W_WIN1 = 1280
W_WIN2 = 2560


def _extract_topk(d2_ref, idx_ref, r, width, off):
    iota = jax.lax.broadcasted_iota(jnp.int32, (r, width), 1)
    iota_k = jax.lax.broadcasted_iota(jnp.int32, (r, K), 1)

    def body(t, idxs):
        d = d2_ref[:, :width]
        m = jnp.min(d, axis=1, keepdims=True)
        cand = jnp.where(d <= m, iota, width)
        j = jnp.min(cand, axis=1, keepdims=True)
        d2_ref[:, :width] = jnp.where(iota == j, jnp.inf, d)
        return jnp.where(iota_k == t, j, idxs)

    idx_ref[...] = jax.lax.fori_loop(
        0, K, body, jnp.zeros((r, K), jnp.int32)) + off


def _topk_body(x_ref, xt_ref, bat_ref, batt_ref, idx_ref, d2_ref):
    r = x_ref.shape[0]
    n = xt_ref.shape[1]
    xb = x_ref[...]
    sqr = jnp.sum(xb * xb, axis=1, keepdims=True)
    bat = bat_ref[...]
    batt = batt_ref[...]
    iota_n = jax.lax.broadcasted_iota(jnp.int32, (1, n), 1)

    bmin = jnp.min(bat)
    bmax = jnp.max(bat)
    lo = jnp.min(jnp.where(batt == bmin, iota_n, n))
    hi = jnp.max(jnp.where(batt == bmax, iota_n, -1)) + 1

    def _windowed(width):
        lo_al = jnp.minimum((lo // 256) * 256, n - width)
        lo_al = pl.multiple_of(lo_al, 256)
        return lo_al, (hi - lo_al) <= width

    lo1, fits1 = _windowed(W_WIN1)
    lo2, fits2 = _windowed(W_WIN2)

    def _run(width, lo_al):
        xt = xt_ref[:, pl.ds(lo_al, width)]
        sqc = jnp.sum(xt * xt, axis=0, keepdims=True)
        dot = _mdot(xb, xt)
        d2 = sqr + sqc - 2.0 * dot
        mask = bat != batt_ref[:, pl.ds(lo_al, width)]
        d2_ref[:, :width] = jnp.where(mask, jnp.inf, d2)
        _extract_topk(d2_ref, idx_ref, r, width, lo_al)

    @pl.when(fits1)
    def _():
        _run(W_WIN1, lo1)

    @pl.when(jnp.logical_not(fits1) & fits2)
    def _():
        _run(W_WIN2, lo2)

    @pl.when(jnp.logical_not(fits2))
    def _():
        xt = xt_ref[...]
        sqc = jnp.sum(xt * xt, axis=0, keepdims=True)
        dot = _mdot(xb, xt)
        d2 = sqr + sqc - 2.0 * dot
        d2_ref[...] = jnp.where(bat != batt, jnp.inf, d2)
        _extract_topk(d2_ref, idx_ref, r, n, 0)


def _topk(x, xt, bat, batt, block_rows, interpret=False):
    n, d = x.shape
    grid = n // block_rows
    return pl.pallas_call(
        _topk_body,
        grid=(grid,),
        in_specs=[
            pl.BlockSpec((block_rows, d), lambda b: (b, 0)),
            pl.BlockSpec((d, n), lambda b: (0, 0)),
            pl.BlockSpec((block_rows, 1), lambda b: (b, 0)),
            pl.BlockSpec((1, n), lambda b: (0, 0)),
        ],
        out_specs=pl.BlockSpec((block_rows, K), lambda b: (b, 0)),
        out_shape=jax.ShapeDtypeStruct((n, K), jnp.int32),
        scratch_shapes=[pltpu.VMEM((block_rows, n), jnp.float32)],
        interpret=interpret,
    )(x, xt, bat, batt)


def _tc_gather(src, idx_flat, interpret=False):
    return jnp.take(src, idx_flat[0], axis=0)


def _sc_gather(src, idx_flat, window=128, interpret=False):
    m = idx_flat.shape[1]
    dim = src.shape[1]
    mesh = plsc.VectorSubcoreMesh(
        core_axis_name="core", subcore_axis_name="subcore")

    @functools.partial(
        pl.kernel,
        out_type=jax.ShapeDtypeStruct((m, dim), src.dtype),
        mesh=mesh,
        compiler_params=pltpu.CompilerParams(use_tc_tiling_on_sc=False))
    def _k(x_hbm, i_hbm, o_hbm):
        def body(i_vmem, o_vmem):
            pltpu.sync_copy(x_hbm.at[i_vmem.at[0]], o_vmem)

        pltpu.emit_pipeline(
            body,
            grid=(m // window,),
            in_specs=[pl.BlockSpec((1, window), index_map=lambda i: (0, i))],
            out_specs=[pl.BlockSpec((window, dim), index_map=lambda i: (i, 0))],
            core_axis_name=("core", "subcore"),
            dimension_semantics=(pltpu.PARALLEL,),
        )(i_hbm, o_hbm)

    return _k(src, idx_flat)


def _edge_body(x_ref, g_ref, w1_ref, b1_ref, w2_ref, b2_ref, out_ref, *, d):
    r = x_ref.shape[0]
    f2 = w2_ref.shape[1]
    xi = x_ref[...][:, :d]
    xj = g_ref[...].reshape(r, K, g_ref.shape[1])[:, :, :d]
    xi3 = jnp.broadcast_to(xi[:, None, :], (r, K, d))
    e = jnp.concatenate([xi3, xj - xi3], axis=-1)
    e = e.reshape(r * K, 2 * d)
    h = jnp.maximum(_mdot(e, w1_ref[...]) + b1_ref[...], 0.0)
    h = jnp.maximum(_mdot(h, w2_ref[...]) + b2_ref[...], 0.0)
    out = jnp.max(h.reshape(r, K, f2), axis=1)
    out_ref[...] = jnp.maximum(out, 0.0)


def _edge_seg_body(x_ref, g_ref, w1_ref, b1_ref, w2_ref, b2_ref, bat_ref,
                   out_ref, seg_ref, *, d):
    _edge_body(x_ref, g_ref, w1_ref, b1_ref, w2_ref, b2_ref, out_ref, d=d)

    @pl.when(pl.program_id(0) == 0)
    def _():
        seg_ref[...] = jnp.full_like(seg_ref, NEG_INF)

    x2 = out_ref[...]
    bat = bat_ref[...]
    for s in range(B):
        v = jnp.max(jnp.where(bat == s, x2, NEG_INF), axis=0,
                    keepdims=True)
        seg_ref[s:s + 1, :] = jnp.maximum(seg_ref[s:s + 1, :], v)


def _edge_mlp(x, g, w1, b1, w2, b2, bat, d, block_rows, interpret=False):
    n = x.shape[0]
    f1 = w1.shape[1]
    f2 = w2.shape[1]
    grid = n // block_rows
    b1r = b1.reshape(1, f1)
    b2r = b2.reshape(1, f2)
    return pl.pallas_call(
        functools.partial(_edge_seg_body, d=d),
        grid=(grid,),
        in_specs=[
            pl.BlockSpec((block_rows, x.shape[1]), lambda b: (b, 0)),
            pl.BlockSpec((block_rows * K, g.shape[1]), lambda b: (b, 0)),
            pl.BlockSpec(w1.shape, lambda b: (0, 0)),
            pl.BlockSpec(b1r.shape, lambda b: (0, 0)),
            pl.BlockSpec(w2.shape, lambda b: (0, 0)),
            pl.BlockSpec(b2r.shape, lambda b: (0, 0)),
            pl.BlockSpec((block_rows, 1), lambda b: (b, 0)),
        ],
        out_specs=[
            pl.BlockSpec((block_rows, f2), lambda b: (b, 0)),
            pl.BlockSpec((B, f2), lambda b: (0, 0)),
        ],
        out_shape=[
            jax.ShapeDtypeStruct((n, f2), jnp.float32),
            jax.ShapeDtypeStruct((B, f2), jnp.float32),
        ],
        interpret=interpret,
    )(x, g, w1, b1r, w2, b2r, bat)


def _head_body(x1_ref, x2_ref, g_ref, w1_ref, b1_ref, w2_ref, b2_ref,
               w3_ref, b3_ref, x4_ref, score_ref):
    r = x1_ref.shape[0]
    g = jnp.broadcast_to(g_ref[0], (r, g_ref.shape[2]))
    x4 = jnp.concatenate([x1_ref[...], x2_ref[...], g], axis=-1)
    x4_ref[...] = x4
    h = jnp.maximum(_mdot(x4, w1_ref[...]) + b1_ref[...], 0.0)
    h = _mdot(h, w2_ref[...]) + b2_ref[...]
    fea = _mdot(h, w3_ref[...]) + b3_ref[...]
    score_ref[...] = jax.nn.sigmoid(fea)


def _head(x1, x2, gseg3, wh1, bh1, wh2, bh2, wh3, bh3, block_rows,
          interpret=False):
    n = x1.shape[0]
    seg_rep = 1024
    b1r = bh1.reshape(1, -1)
    b2r = bh2.reshape(1, -1)
    b3r = bh3.reshape(1, -1)
    grid = n // block_rows
    return pl.pallas_call(
        _head_body,
        grid=(grid,),
        in_specs=[
            pl.BlockSpec((block_rows, x1.shape[1]), lambda b: (b, 0)),
            pl.BlockSpec((block_rows, x2.shape[1]), lambda b: (b, 0)),
            pl.BlockSpec((1, 1, gseg3.shape[2]),
                         lambda b: (b * block_rows // seg_rep, 0, 0)),
            pl.BlockSpec(wh1.shape, lambda b: (0, 0)),
            pl.BlockSpec(b1r.shape, lambda b: (0, 0)),
            pl.BlockSpec(wh2.shape, lambda b: (0, 0)),
            pl.BlockSpec(b2r.shape, lambda b: (0, 0)),
            pl.BlockSpec(wh3.shape, lambda b: (0, 0)),
            pl.BlockSpec(b3r.shape, lambda b: (0, 0)),
        ],
        out_specs=[
            pl.BlockSpec((block_rows, x1.shape[1] + x2.shape[1]
                          + gseg3.shape[2]), lambda b: (b, 0)),
            pl.BlockSpec((block_rows, 1), lambda b: (b, 0)),
        ],
        out_shape=[
            jax.ShapeDtypeStruct(
                (n, x1.shape[1] + x2.shape[1] + gseg3.shape[2]), jnp.float32),
            jax.ShapeDtypeStruct((n, 1), jnp.float32),
        ],
        interpret=interpret,
    )(x1, x2, gseg3, wh1, b1r, wh2, b2r, wh3, b3r)


def kernel(x, pos, batch, W1a, b1a, W2a, b2a, W1b, b1b, W2b, b2b,
           Wh1, bh1, Wh2, bh2, Wh3, bh3):
    del pos
    n = x.shape[0]
    bat = batch.astype(jnp.int32).reshape(n, 1)
    batt = batch.astype(jnp.int32).reshape(1, n)

    xpad = jnp.pad(x, ((0, 0), (0, 16 - x.shape[1])))
    idx_a = _topk(x, x.T, bat, batt, block_rows=256)
    ga = _sc_gather(xpad, idx_a.reshape(1, n * K))
    x1, _ = _edge_mlp(xpad, ga, W1a, b1a, W2a, b2a, bat, d=3,
                      block_rows=256)

    idx_b = _topk(x1, x1.T, bat, batt, block_rows=256)
    gb = _sc_gather(x1, idx_b.reshape(1, n * K), window=256)
    x2, gseg = _edge_mlp(x1, gb, W1b, b1b, W2b, b2b, bat, d=64,
                         block_rows=256)

    x4, score = _head(x1, x2, gseg.reshape(B, 1, -1),
                      Wh1, bh1, Wh2, bh2, Wh3, bh3, block_rows=512)
    return (x4, score)
